# baseline (speedup 1.0000x reference)
import functools

import jax
import jax.numpy as jnp
from jax import lax
from jax.experimental import pallas as pl
from jax.experimental.pallas import tpu as pltpu
from jax.experimental.pallas import tpu_sc as plsc

_NC = 2
_NS = 16
_LANES = 16


def _sc_dots(inp, tgt2d, emb_table, lin_weight, B, K, D):
    NW = _NC * _NS
    BPW = B // NW
    CB = 64
    NSUB = BPW // CB
    PR = CB * K
    RW = PR // 128

    mesh = plsc.VectorSubcoreMesh(
        core_axis_name="c", subcore_axis_name="s",
        num_cores=_NC, num_subcores=_NS)

    @functools.partial(
        pl.kernel,
        mesh=mesh,
        compiler_params=pltpu.CompilerParams(
            use_tc_tiling_on_sc=False, needs_layout_passes=False),
        out_type=jax.ShapeDtypeStruct((B * K,), jnp.float32),
        scratch_types=[
            [pltpu.VMEM((CB,), jnp.int32)] * 2,
            [pltpu.VMEM((PR,), jnp.int32)] * 2,
            [pltpu.VMEM((CB, D), jnp.bfloat16)] * 2,
            [pltpu.VMEM((PR, D), jnp.bfloat16)] * 2,
            pltpu.VMEM((PR,), jnp.float32),
            pltpu.VMEM((4 * K, _LANES), jnp.float32),
            [pltpu.SemaphoreType.DMA] * 2,
        ],
    )
    def sc_kernel(inp_hbm, tgt_hbm, emb_hbm, lin_hbm, out_hbm,
                  idx_b, tgt_b, e_b, t_b, dots_v, macc_v, sems):
        wid = lax.axis_index("s") * _NC + lax.axis_index("c")
        NCH = D // (2 * _LANES)

        def unpk(ref, row, c):
            ab = ref[row, pl.ds(c * 2 * _LANES, 2 * _LANES)]
            return plsc.unpack(ab, format=plsc.PackFormat.INTERLEAVED,
                               preferred_element_type=jnp.float32)

        def issue(s, bi):
            base_b = wid * BPW + s * CB
            pltpu.sync_copy(inp_hbm.at[pl.ds(base_b, CB)], idx_b[bi])
            pltpu.sync_copy(tgt_hbm.at[pl.ds(base_b * K, PR)], tgt_b[bi])
            pltpu.async_copy(emb_hbm.at[idx_b[bi]], e_b[bi], sems[bi])
            pltpu.async_copy(lin_hbm.at[tgt_b[bi]], t_b[bi], sems[bi])

        def drain(bi):
            pltpu.make_async_copy(emb_hbm.at[idx_b[bi]], e_b[bi], sems[bi]).wait()
            pltpu.make_async_copy(lin_hbm.at[tgt_b[bi]], t_b[bi], sems[bi]).wait()

        def compute(s, bi):
            e_v, t_v = e_b[bi], t_b[bi]

            def bblock(q, carry2):
                b0 = q * 4
                es = [[unpk(e_v, b0 + bb, c) for c in range(NCH)]
                      for bb in range(4)]
                for bb in range(4):
                    for k in range(K):
                        row = (b0 + bb) * K + k
                        acc = None
                        for c in range(NCH):
                            t0, t1 = unpk(t_v, row, c)
                            e0, e1 = es[bb][c]
                            pp = t0 * e0 + t1 * e1
                            acc = pp if acc is None else acc + pp
                        macc_v[bb * K + k] = acc
                for j in range(4 * K // _LANES):
                    rowsel = jnp.arange(_LANES, dtype=jnp.int32) + j * _LANES
                    dot = plsc.load_gather(
                        macc_v, [rowsel, jnp.full((_LANES,), 0, jnp.int32)])
                    for c in range(1, _LANES):
                        dot = dot + plsc.load_gather(
                            macc_v, [rowsel, jnp.full((_LANES,), c, jnp.int32)])
                    dots_v[pl.ds(q * 4 * K + j * _LANES, _LANES)] = dot
                return carry2

            lax.fori_loop(0, CB // 4, bblock, 0)
            base_b = wid * BPW + s * CB
            pltpu.sync_copy(dots_v, out_hbm.at[pl.ds(base_b * K, PR)])

        issue(0, 0)

        def half(h, carry):
            issue(2 * h + 1, 1)
            drain(0)
            compute(2 * h, 0)

            @pl.when(h < NSUB // 2 - 1)
            def _():
                issue(2 * h + 2, 0)

            drain(1)
            compute(2 * h + 1, 1)
            return carry

        lax.fori_loop(0, NSUB // 2, half, 0)

    return sc_kernel(inp, tgt2d, emb_table, lin_weight)


def _tc_loss_sum(dots2d):
    def body(x_ref, o_ref):
        x = x_ref[...]
        o_ref[0, 0] = jnp.sum(jnp.log1p(jnp.exp(-x)))

    return pl.pallas_call(
        body,
        out_shape=jax.ShapeDtypeStruct((1, 1), jnp.float32),
        out_specs=pl.BlockSpec(memory_space=pltpu.SMEM),
    )(dots2d)


def kernel(input, targets, emb_table, lin_weight):
    B, = input.shape
    _, K = targets.shape
    _, D = emb_table.shape
    inp = input.astype(jnp.int32)
    tgt_flat = targets.astype(jnp.int32).reshape(B * K)
    emb_bf = emb_table.astype(jnp.bfloat16)
    lin_bf = lin_weight.astype(jnp.bfloat16)
    dots = _sc_dots(inp, tgt_flat, emb_bf, lin_bf, B, K, D)
    s = _tc_loss_sum(dots.reshape(B * K // 128, 128))
    return s[0, 0] / (B * K)
